# baseline (speedup 1.0000x reference)
import jax
import jax.numpy as jnp
from jax.experimental import pallas as pl
from jax.experimental.pallas import tpu as pltpu

_Q = 1024
_D = 128
_K = 100000
_KT = 2000
_NT = _K // _KT

_BIG = float("inf")


def _knn_body(qm2_ref, k_ref, od_ref, oi_ref, bd_ref, bi_ref):
    t = pl.program_id(0)

    @pl.when(t == 0)
    def _():
        bd_ref[...] = jnp.full((2, _Q), _BIG, jnp.float32)
        bi_ref[...] = jnp.zeros((2, _Q), jnp.float32)

    qm2 = qm2_ref[...]
    k = k_ref[...]
    kq2 = jax.lax.dot_general(k, qm2, (((1,), (0,)), ((), ())),
                              preferred_element_type=jnp.float32)
    ksq = jnp.sum(k * k, axis=1, keepdims=True)
    s = kq2 + ksq

    ridx = jax.lax.broadcasted_iota(jnp.int32, (_KT, _Q), 0).astype(jnp.float32)

    m1 = jnp.min(s, axis=0, keepdims=True)
    i1 = jnp.min(jnp.where(s == m1, ridx, float(_K)), axis=0, keepdims=True)
    not_i1 = ridx != i1
    m2 = jnp.min(jnp.where(not_i1, s, _BIG), axis=0, keepdims=True)
    i2 = jnp.min(jnp.where((s == m2) & not_i1, ridx, float(_K)),
                 axis=0, keepdims=True)
    base = (t * _KT).astype(jnp.float32)
    gi1 = i1 + base
    gi2 = i2 + base

    r1v = bd_ref[0:1, :]
    r2v = bd_ref[1:2, :]
    r1i = bi_ref[0:1, :]
    r2i = bi_ref[1:2, :]

    take_new1 = m1 < r1v
    o1v = jnp.where(take_new1, m1, r1v)
    o1i = jnp.where(take_new1, gi1, r1i)
    o2v = jnp.where(take_new1,
                    jnp.where(m2 < r1v, m2, r1v),
                    jnp.where(m1 < r2v, m1, r2v))
    o2i = jnp.where(take_new1,
                    jnp.where(m2 < r1v, gi2, r1i),
                    jnp.where(m1 < r2v, gi1, r2i))
    bd_ref[0:1, :] = o1v
    bd_ref[1:2, :] = o2v
    bi_ref[0:1, :] = o1i
    bi_ref[1:2, :] = o2i

    @pl.when(t == _NT - 1)
    def _():
        qm2_last = qm2_ref[...]
        qsq = 0.25 * jnp.sum(qm2_last * qm2_last, axis=0, keepdims=True)
        od_ref[0:1, :] = bd_ref[0:1, :] + qsq
        od_ref[1:2, :] = bd_ref[1:2, :] + qsq
        oi_ref[...] = bi_ref[...].astype(jnp.int32)


def kernel(queries, keys):
    qm2 = (-2.0 * queries).T
    od, oi = pl.pallas_call(
        _knn_body,
        grid=(_NT,),
        in_specs=[
            pl.BlockSpec((_D, _Q), lambda t: (0, 0)),
            pl.BlockSpec((_KT, _D), lambda t: (t, 0)),
        ],
        out_specs=[
            pl.BlockSpec((2, _Q), lambda t: (0, 0)),
            pl.BlockSpec((2, _Q), lambda t: (0, 0)),
        ],
        out_shape=[
            jax.ShapeDtypeStruct((2, _Q), jnp.float32),
            jax.ShapeDtypeStruct((2, _Q), jnp.int32),
        ],
        scratch_shapes=[
            pltpu.VMEM((2, _Q), jnp.float32),
            pltpu.VMEM((2, _Q), jnp.float32),
        ],
        compiler_params=pltpu.CompilerParams(
            dimension_semantics=("arbitrary",),
        ),
    )(qm2, keys)
    return od.T, oi.T


# speedup vs baseline: 1.0921x; 1.0921x over previous
import jax
import jax.numpy as jnp
from jax.experimental import pallas as pl
from jax.experimental.pallas import tpu as pltpu

_Q = 1024
_D = 128
_K = 100000
_KT = 2000
_NT = _K // _KT

_BIG = float("inf")


def _knn_body(qm2_ref, k_ref, od_ref, oi_ref, s_scr, bd_ref, bi_ref):
    t = pl.program_id(0)

    @pl.when(t == 0)
    def _():
        bd_ref[...] = jnp.full((2, _Q), _BIG, jnp.float32)
        bi_ref[...] = jnp.zeros((2, _Q), jnp.float32)

    @pl.when(t < _NT)
    def _():
        qm2 = qm2_ref[...]
        k = k_ref[...]
        kq2 = jax.lax.dot_general(k, qm2, (((1,), (0,)), ((), ())),
                                  preferred_element_type=jnp.float32)
        ksq = jnp.sum(k * k, axis=1, keepdims=True)
        s_scr[t % 2] = kq2 + ksq

    @pl.when(t > 0)
    def _():
        s = s_scr[(t - 1) % 2]
        ridx = jax.lax.broadcasted_iota(
            jnp.int32, (_KT, _Q), 0).astype(jnp.float32)

        m1 = jnp.min(s, axis=0, keepdims=True)
        i1 = jnp.min(jnp.where(s == m1, ridx, float(_K)),
                     axis=0, keepdims=True)
        s2 = jnp.where(ridx == i1, _BIG, s)
        m2 = jnp.min(s2, axis=0, keepdims=True)
        i2 = jnp.min(jnp.where(s2 == m2, ridx, float(_K)),
                     axis=0, keepdims=True)
        base = ((t - 1) * _KT).astype(jnp.float32)
        gi1 = i1 + base
        gi2 = i2 + base

        r1v = bd_ref[0:1, :]
        r2v = bd_ref[1:2, :]
        r1i = bi_ref[0:1, :]
        r2i = bi_ref[1:2, :]

        take_new1 = m1 < r1v
        o1v = jnp.where(take_new1, m1, r1v)
        o1i = jnp.where(take_new1, gi1, r1i)
        o2v = jnp.where(take_new1,
                        jnp.where(m2 < r1v, m2, r1v),
                        jnp.where(m1 < r2v, m1, r2v))
        o2i = jnp.where(take_new1,
                        jnp.where(m2 < r1v, gi2, r1i),
                        jnp.where(m1 < r2v, gi1, r2i))
        bd_ref[0:1, :] = o1v
        bd_ref[1:2, :] = o2v
        bi_ref[0:1, :] = o1i
        bi_ref[1:2, :] = o2i

    @pl.when(t == _NT)
    def _():
        qm2_last = qm2_ref[...]
        qsq = 0.25 * jnp.sum(qm2_last * qm2_last, axis=0, keepdims=True)
        od_ref[0:1, :] = bd_ref[0:1, :] + qsq
        od_ref[1:2, :] = bd_ref[1:2, :] + qsq
        oi_ref[...] = bi_ref[...].astype(jnp.int32)


def kernel(queries, keys):
    qm2 = (-2.0 * queries).T
    od, oi = pl.pallas_call(
        _knn_body,
        grid=(_NT + 1,),
        in_specs=[
            pl.BlockSpec((_D, _Q), lambda t: (0, 0)),
            pl.BlockSpec((_KT, _D), lambda t: (jnp.minimum(t, _NT - 1), 0)),
        ],
        out_specs=[
            pl.BlockSpec((2, _Q), lambda t: (0, 0)),
            pl.BlockSpec((2, _Q), lambda t: (0, 0)),
        ],
        out_shape=[
            jax.ShapeDtypeStruct((2, _Q), jnp.float32),
            jax.ShapeDtypeStruct((2, _Q), jnp.int32),
        ],
        scratch_shapes=[
            pltpu.VMEM((2, _KT, _Q), jnp.float32),
            pltpu.VMEM((2, _Q), jnp.float32),
            pltpu.VMEM((2, _Q), jnp.float32),
        ],
        compiler_params=pltpu.CompilerParams(
            dimension_semantics=("arbitrary",),
        ),
    )(qm2, keys)
    return od.T, oi.T


# speedup vs baseline: 1.1132x; 1.0193x over previous
import jax
import jax.numpy as jnp
from jax.experimental import pallas as pl
from jax.experimental.pallas import tpu as pltpu

_Q = 1024
_D = 128
_K = 100000
_KT = 4000
_NT = _K // _KT

_BIG = float("inf")


def _knn_body(qm2_ref, k_ref, od_ref, oi_ref, s_scr, bd_ref, bi_ref):
    t = pl.program_id(0)

    @pl.when(t == 0)
    def _():
        bd_ref[...] = jnp.full((2, _Q), _BIG, jnp.float32)
        bi_ref[...] = jnp.zeros((2, _Q), jnp.float32)

    @pl.when(t < _NT)
    def _():
        qm2 = qm2_ref[...]
        k = k_ref[...]
        kq2 = jax.lax.dot_general(k, qm2, (((1,), (0,)), ((), ())),
                                  preferred_element_type=jnp.float32)
        ksq = jnp.sum(k * k, axis=1, keepdims=True)
        s_scr[t % 2] = kq2 + ksq

    @pl.when(t > 0)
    def _():
        s = s_scr[(t - 1) % 2]
        ridx = jax.lax.broadcasted_iota(
            jnp.int32, (_KT, _Q), 0).astype(jnp.float32)

        m1 = jnp.min(s, axis=0, keepdims=True)
        i1 = jnp.min(jnp.where(s == m1, ridx, float(_K)),
                     axis=0, keepdims=True)
        s2 = jnp.where(ridx == i1, _BIG, s)
        m2 = jnp.min(s2, axis=0, keepdims=True)
        i2 = jnp.min(jnp.where(s2 == m2, ridx, float(_K)),
                     axis=0, keepdims=True)
        base = ((t - 1) * _KT).astype(jnp.float32)
        gi1 = i1 + base
        gi2 = i2 + base

        r1v = bd_ref[0:1, :]
        r2v = bd_ref[1:2, :]
        r1i = bi_ref[0:1, :]
        r2i = bi_ref[1:2, :]

        take_new1 = m1 < r1v
        o1v = jnp.where(take_new1, m1, r1v)
        o1i = jnp.where(take_new1, gi1, r1i)
        o2v = jnp.where(take_new1,
                        jnp.where(m2 < r1v, m2, r1v),
                        jnp.where(m1 < r2v, m1, r2v))
        o2i = jnp.where(take_new1,
                        jnp.where(m2 < r1v, gi2, r1i),
                        jnp.where(m1 < r2v, gi1, r2i))
        bd_ref[0:1, :] = o1v
        bd_ref[1:2, :] = o2v
        bi_ref[0:1, :] = o1i
        bi_ref[1:2, :] = o2i

    @pl.when(t == _NT)
    def _():
        qm2_last = qm2_ref[...]
        qsq = 0.25 * jnp.sum(qm2_last * qm2_last, axis=0, keepdims=True)
        od_ref[0:1, :] = bd_ref[0:1, :] + qsq
        od_ref[1:2, :] = bd_ref[1:2, :] + qsq
        oi_ref[...] = bi_ref[...].astype(jnp.int32)


def kernel(queries, keys):
    qm2 = (-2.0 * queries).T
    od, oi = pl.pallas_call(
        _knn_body,
        grid=(_NT + 1,),
        in_specs=[
            pl.BlockSpec((_D, _Q), lambda t: (0, 0)),
            pl.BlockSpec((_KT, _D), lambda t: (jnp.minimum(t, _NT - 1), 0)),
        ],
        out_specs=[
            pl.BlockSpec((2, _Q), lambda t: (0, 0)),
            pl.BlockSpec((2, _Q), lambda t: (0, 0)),
        ],
        out_shape=[
            jax.ShapeDtypeStruct((2, _Q), jnp.float32),
            jax.ShapeDtypeStruct((2, _Q), jnp.int32),
        ],
        scratch_shapes=[
            pltpu.VMEM((2, _KT, _Q), jnp.float32),
            pltpu.VMEM((2, _Q), jnp.float32),
            pltpu.VMEM((2, _Q), jnp.float32),
        ],
        compiler_params=pltpu.CompilerParams(
            dimension_semantics=("arbitrary",),
        ),
    )(qm2, keys)
    return od.T, oi.T
